# baseline (speedup 1.0000x reference)
import functools

import jax
import jax.numpy as jnp
from jax import lax
from jax.experimental import pallas as pl
from jax.experimental.pallas import tpu as pltpu
from jax.experimental.pallas import tpu_sc as plsc

N = 10000
E = 320000
NP = 10240
C = 128
NT = 32
NBUF = 4
CT = 80
EPAD = NT * CT * C - E
RPT = NP // 16
NB = NP

f32 = jnp.float32
i32 = jnp.int32

_MESH = plsc.VectorSubcoreMesh(core_axis_name="c", subcore_axis_name="s")


@functools.cache
def _hist_kernel():
    @functools.partial(
        pl.kernel,
        out_type=jax.ShapeDtypeStruct((NT, 8, NB), f32),
        mesh=_MESH,
        scratch_types=[
            pltpu.VMEM((CT, C), i32),
            pltpu.VMEM((NB,), f32),
        ],
        compiler_params=pltpu.CompilerParams(needs_layout_passes=False),
    )
    def hist(idx8_hbm, out_hbm, idx_v, hist_v):
        cc = lax.axis_index("c")
        ss = lax.axis_index("s")
        wid = cc * 16 + ss
        zero16 = jnp.zeros((16,), f32)
        ones16 = jnp.ones((16,), f32)
        for g in range(8):
            def zb(i, _):
                hist_v[pl.ds(i * 16, 16)] = zero16
                return 0
            lax.fori_loop(0, NB // 16, zb, 0)
            pltpu.sync_copy(idx8_hbm.at[g, wid], idx_v)

            def eb(j, _):
                for kk in range(C // 16):
                    idx = idx_v[j, pl.ds(kk * 16, 16)]
                    plsc.addupdate_scatter(hist_v, [idx], ones16)
                return 0
            lax.fori_loop(0, CT, eb, 0)
            pltpu.sync_copy(hist_v, out_hbm.at[wid, g])

    return hist


@functools.cache
def _apply_packed_kernel(H):
    NB2 = 2

    @functools.partial(
        pl.kernel,
        out_type=jax.ShapeDtypeStruct((2, NP, H), f32),
        mesh=_MESH,
        scratch_types=[
            pltpu.VMEM((CT, C), i32),
            pltpu.VMEM((C,), i32),
            pltpu.VMEM((C,), i32),
            pltpu.VMEM((C,), i32),
            pltpu.VMEM((C,), i32),
            pltpu.VMEM((C, H), f32),
            pltpu.VMEM((C, H), f32),
            pltpu.VMEM_SHARED((NP, H), f32),
            pltpu.SemaphoreType.DMA,
            pltpu.SemaphoreType.DMA,
        ],
        compiler_params=pltpu.CompilerParams(use_tc_tiling_on_sc=False,
                                             needs_layout_passes=False),
    )
    def apply(pk_hbm, hs_hbm, out_hbm, pk_v, si0, si1, di0, di1,
              r0, r1, acc, s0, s1):
        sidx = (si0, si1)
        didx = (di0, di1)
        rows = (r0, r1)
        sems = (s0, s1)
        cc = lax.axis_index("c")
        ss = lax.axis_index("s")
        wid = cc * 16 + ss
        base = ss * RPT

        pltpu.sync_copy(pk_hbm.at[wid], pk_v)

        zero16 = jnp.zeros((16,), f32)
        def zrow(i, _):
            for kk in range(H // 16):
                r0[i, pl.ds(kk * 16, 16)] = zero16
            return 0
        lax.fori_loop(0, C, zrow, 0)
        for b in range(RPT // C):
            pltpu.sync_copy(r0, acc.at[pl.ds(base + b * C, C)])

        mask = jnp.full((16,), (1 << 14) - 1, i32)
        sh14 = jnp.full((16,), 14, i32)

        def unpack(j, b):
            for kk in range(C // 16):
                p = pk_v[j, pl.ds(kk * 16, 16)]
                sidx[b][pl.ds(kk * 16, 16)] = jnp.bitwise_and(p, mask)
                didx[b][pl.ds(kk * 16, 16)] = lax.shift_right_logical(p, sh14)

        for b in range(NB2):
            unpack(b, b)
            pltpu.async_copy(hs_hbm.at[sidx[b]], rows[b], sems[b])
        plsc.subcore_barrier()

        def body(k, _):
            j0 = k * NB2
            for b in range(NB2):
                pltpu.make_async_copy(hs_hbm.at[sidx[b]],
                                      rows[b], sems[b]).wait()
                pltpu.sync_copy(rows[b], acc.at[didx[b]], add=True)
                unpack(j0 + b + NB2, b)
                pltpu.async_copy(hs_hbm.at[sidx[b]], rows[b], sems[b])
            return 0
        lax.fori_loop(0, CT // NB2 - 1, body, 0)

        for b in range(NB2):
            pltpu.make_async_copy(hs_hbm.at[sidx[b]],
                                  rows[b], sems[b]).wait()
            pltpu.sync_copy(rows[b], acc.at[didx[b]], add=True)

        plsc.subcore_barrier()
        pltpu.sync_copy(acc.at[pl.ds(base, RPT)],
                        out_hbm.at[cc, pl.ds(base, RPT)])

    return apply


@functools.cache
def _apply_kernel(H):
    @functools.partial(
        pl.kernel,
        out_type=jax.ShapeDtypeStruct((2, NP, H), f32),
        mesh=_MESH,
        scratch_types=[
            pltpu.VMEM((CT, C), i32),
            pltpu.VMEM((CT, C), i32),
            pltpu.VMEM((C, H), f32),
            pltpu.VMEM((C, H), f32),
            pltpu.VMEM((C, H), f32),
            pltpu.VMEM((C, H), f32),
            pltpu.VMEM_SHARED((NP, H), f32),
            pltpu.SemaphoreType.DMA,
            pltpu.SemaphoreType.DMA,
            pltpu.SemaphoreType.DMA,
            pltpu.SemaphoreType.DMA,
        ],
        compiler_params=pltpu.CompilerParams(use_tc_tiling_on_sc=False),
    )
    def apply(src_hbm, dst_hbm, hs_hbm, out_hbm, src_v, dst_v,
              r0, r1, r2, r3, acc, s0, s1, s2, s3):
        rows = (r0, r1, r2, r3)
        sems = (s0, s1, s2, s3)
        cc = lax.axis_index("c")
        ss = lax.axis_index("s")
        wid = cc * 16 + ss
        base = ss * RPT

        pltpu.sync_copy(src_hbm.at[wid], src_v)
        pltpu.sync_copy(dst_hbm.at[wid], dst_v)

        zero16 = jnp.zeros((16,), f32)
        def zrow(i, _):
            for kk in range(H // 16):
                r0[i, pl.ds(kk * 16, 16)] = zero16
            return 0
        lax.fori_loop(0, C, zrow, 0)
        for b in range(RPT // C):
            pltpu.sync_copy(r0, acc.at[pl.ds(base + b * C, C)])

        for b in range(NBUF):
            pltpu.async_copy(hs_hbm.at[src_v.at[b]], rows[b], sems[b])
        plsc.subcore_barrier()

        def body(k, _):
            j0 = k * NBUF
            for b in range(NBUF):
                j = j0 + b
                pltpu.make_async_copy(hs_hbm.at[src_v.at[j]],
                                      rows[b], sems[b]).wait()
                pltpu.sync_copy(rows[b], acc.at[dst_v.at[j]], add=True)
                pltpu.async_copy(hs_hbm.at[src_v.at[j + NBUF]],
                                 rows[b], sems[b])
            return 0
        lax.fori_loop(0, CT // NBUF - 1, body, 0)

        for b in range(NBUF):
            j = CT - NBUF + b
            pltpu.make_async_copy(hs_hbm.at[src_v.at[j]],
                                  rows[b], sems[b]).wait()
            pltpu.sync_copy(rows[b], acc.at[dst_v.at[j]], add=True)

        plsc.subcore_barrier()
        pltpu.sync_copy(acc.at[pl.ds(base, RPT)],
                        out_hbm.at[cc, pl.ds(base, RPT)])

    return apply


_BV = 1000


def _norms(hists):
    BB = 1280

    def body(h_ref, o_ref):
        v = h_ref[...]
        deg = v[0]
        for t in range(1, NT):
            deg = deg + v[t]
        row = lax.broadcasted_iota(i32, (8, BB), 0)
        deg = deg + jnp.where(row >= 6, 1.0, 0.0).astype(f32)
        o_ref[...] = jnp.where(deg > 0, lax.rsqrt(deg), 0.0)

    return pl.pallas_call(
        body,
        grid=(NB // BB,),
        in_specs=[pl.BlockSpec((NT, 8, BB), lambda i: (0, 0, i))],
        out_specs=pl.BlockSpec((8, BB), lambda i: (0, i)),
        out_shape=jax.ShapeDtypeStruct((8, NB), f32),
    )(hists)


def _mm_scale(x, W, ns):
    K, H2 = W.shape

    def body(x_ref, w_ref, ns_ref, o_ref):
        h = jnp.dot(x_ref[...], w_ref[...], preferred_element_type=f32,
                    precision=lax.Precision.HIGHEST)
        o_ref[...] = h * ns_ref[...]

    return pl.pallas_call(
        body,
        grid=(N // _BV,),
        in_specs=[pl.BlockSpec((_BV, K), lambda i: (i, 0)),
                  pl.BlockSpec((K, H2), lambda i: (0, 0)),
                  pl.BlockSpec((_BV, 1), lambda i: (i, 0))],
        out_specs=pl.BlockSpec((_BV, H2), lambda i: (i, 0)),
        out_shape=jax.ShapeDtypeStruct((N, H2), f32),
    )(x, W, ns)


def _combine(parts, extra, nd, b, act, W=None, ns=None):
    H = parts.shape[2]
    have_extra = extra is not None
    have_mm = W is not None

    def body(*refs):
        it = iter(refs)
        p_ref = next(it)
        e_ref = next(it) if have_extra else None
        nd_ref = next(it)
        b_ref = next(it)
        w_ref = next(it) if have_mm else None
        ns_ref = next(it) if have_mm else None
        o_ref = next(it)
        v = p_ref[...]
        agg = v[0] + v[1]
        if have_extra:
            agg = agg + e_ref[...]
        h = agg * nd_ref[...] + b_ref[...]
        if act:
            h = jnp.maximum(h, 0.0)
        if have_mm:
            h = jnp.dot(h, w_ref[...], preferred_element_type=f32,
                        precision=lax.Precision.HIGHEST) * ns_ref[...]
        o_ref[...] = h

    H2 = W.shape[1] if have_mm else H
    in_specs = [pl.BlockSpec((2, _BV, H), lambda i: (0, i, 0))]
    args = [parts]
    if have_extra:
        in_specs.append(pl.BlockSpec((_BV, H), lambda i: (i, 0)))
        args.append(extra)
    in_specs += [pl.BlockSpec((_BV, 1), lambda i: (i, 0)),
                 pl.BlockSpec((1, H), lambda i: (0, 0))]
    args += [nd, b.reshape(1, H)]
    if have_mm:
        in_specs += [pl.BlockSpec(W.shape, lambda i: (0, 0)),
                     pl.BlockSpec((_BV, 1), lambda i: (i, 0))]
        args += [W, ns]

    return pl.pallas_call(
        body,
        grid=(N // _BV,),
        in_specs=in_specs,
        out_specs=pl.BlockSpec((_BV, H2), lambda i: (i, 0)),
        out_shape=jax.ShapeDtypeStruct((N, H2), f32),
    )(*args)


def _fusion(h0, h1, h2, Wf1, Wf2, Wf3, WfcT, bfc, Wma, ns):
    H = h0.shape[1]

    def body(h0_ref, h1_ref, h2_ref, w1_ref, w2_ref, w3_ref, wc_ref, bc_ref,
             wm_ref, ns_ref, o_ref):
        kw = dict(preferred_element_type=f32, precision=lax.Precision.HIGHEST)
        y = (jnp.dot(h0_ref[...], w1_ref[...], **kw)
             + jnp.dot(h1_ref[...], w2_ref[...], **kw)
             + jnp.dot(h2_ref[...], w3_ref[...], **kw))
        xh0 = jnp.dot(y, wc_ref[...], **kw) + bc_ref[...]
        o_ref[...] = jnp.dot(xh0, wm_ref[...], **kw) * ns_ref[...]

    mat = pl.BlockSpec((H, H), lambda i: (0, 0))
    blk = pl.BlockSpec((_BV, H), lambda i: (i, 0))
    return pl.pallas_call(
        body,
        grid=(N // _BV,),
        in_specs=[blk, blk, blk, mat, mat, mat, mat,
                  pl.BlockSpec((1, H), lambda i: (0, 0)),
                  mat,
                  pl.BlockSpec((_BV, 1), lambda i: (i, 0))],
        out_specs=blk,
        out_shape=jax.ShapeDtypeStruct((N, H), f32),
    )(h0, h1, h2, Wf1, Wf2, Wf3, WfcT, bfc.reshape(1, H), Wma, ns)


def _decoder(xh, Wdec):
    BR = 200
    H = xh.shape[1]

    def body(xr_ref, xf_ref, w_ref, o_ref):
        kw = dict(preferred_element_type=f32, precision=lax.Precision.HIGHEST)
        t = jnp.dot(xr_ref[...], w_ref[...], **kw)
        logits = lax.dot_general(t, xf_ref[...], (((1,), (1,)), ((), ())), **kw)
        o_ref[...] = 1.0 / (1.0 + jnp.exp(-logits))

    return pl.pallas_call(
        body,
        grid=(N // BR,),
        in_specs=[pl.BlockSpec((BR, H), lambda i: (i, 0)),
                  pl.BlockSpec((N, H), lambda i: (0, 0)),
                  pl.BlockSpec((H, H), lambda i: (0, 0))],
        out_specs=pl.BlockSpec((BR, N), lambda i: (i, 0)),
        out_shape=jax.ShapeDtypeStruct((N, N), f32),
    )(xh, xh, Wdec)


def _prep_edges(g):
    src = jnp.concatenate([g[0].astype(i32), jnp.zeros((EPAD,), i32)])
    srch = jnp.concatenate([g[0].astype(i32), jnp.full((EPAD,), N, i32)])
    dst = jnp.concatenate([g[1].astype(i32), jnp.full((EPAD,), N, i32)])
    pk = jnp.bitwise_or(src, jnp.left_shift(dst, 14))
    return (src.reshape(NT, CT, C), srch.reshape(NT, CT, C),
            dst.reshape(NT, CT, C), pk.reshape(NT, CT, C))


def kernel(graph0, graph1, graph2, feature0, feature1, feature2, graph,
           W0a, b0a, W0b, b0b, W1a, b1a, W1b, b1b, W2a, b2a, W2b, b2b,
           Wma, bma, Wmb, bmb, Wf1, Wf2, Wf3, Wfc, bfc, Wdec):
    s0, sh0, d0, p0 = _prep_edges(graph0)
    s1, sh1, d1, p1 = _prep_edges(graph1)
    s2, sh2, d2, p2 = _prep_edges(graph2)
    sm, shm, dm, _ = _prep_edges(graph)

    idx8 = jnp.stack([sh0, d0, sh1, d1, sh2, d2, shm, dm])
    norms = _norms(_hist_kernel()(idx8))

    def col(g):
        return norms[g, :N].reshape(N, 1)

    ns0, nd0, ns1, nd1, ns2, nd2, nsm, ndm = (col(g) for g in range(8))

    ap128 = _apply_packed_kernel(128)
    ap64 = _apply_kernel(64)
    ap32 = _apply_kernel(32)

    def view(x, s, d, pk, ns, nd, Wa, ba, Wb, bb):
        hs = _mm_scale(x, Wa, ns)
        parts = ap128(pk, hs)
        hs2 = _combine(parts, None, nd, ba, True, W=Wb, ns=ns)
        parts2 = ap64(s, d, hs2)
        return _combine(parts2, None, nd, bb, False)

    h0 = view(feature0, s0, d0, p0, ns0, nd0, W0a, b0a, W0b, b0b)
    h1 = view(feature1, s1, d1, p1, ns1, nd1, W1a, b1a, W1b, b1b)
    h2 = view(feature2, s2, d2, p2, ns2, nd2, W2a, b2a, W2b, b2b)

    hs_m = _fusion(h0, h1, h2, Wf1, Wf2, Wf3, Wfc.T, bfc, Wma, nsm)
    parts_m = ap64(sm, dm, hs_m)
    hs2_m = _combine(parts_m, hs_m, ndm, bma, True, W=Wmb, ns=nsm)
    parts2_m = ap32(sm, dm, hs2_m)
    xh = _combine(parts2_m, hs2_m, ndm, bmb, False)

    adj0 = _decoder(xh, Wdec)
    return adj0, xh


# speedup vs baseline: 1.2860x; 1.2860x over previous
import functools

import jax
import jax.numpy as jnp
from jax import lax
from jax.experimental import pallas as pl
from jax.experimental.pallas import tpu as pltpu
from jax.experimental.pallas import tpu_sc as plsc

N = 10000
E = 320000
NP = 10240
C = 128
NT = 32
CT = (E + NT * C - 1) // (NT * C)
EPAD = NT * CT * C - E
RPT = NP // 16
NB = NP

f32 = jnp.float32
i32 = jnp.int32

_MESH = plsc.VectorSubcoreMesh(core_axis_name="c", subcore_axis_name="s")


@functools.cache
def _hist_kernel():
    @functools.partial(
        pl.kernel,
        out_type=jax.ShapeDtypeStruct((NT, 8, NB), f32),
        mesh=_MESH,
        scratch_types=[
            pltpu.VMEM((CT, C), i32),
            pltpu.VMEM((NB,), f32),
        ],
        compiler_params=pltpu.CompilerParams(needs_layout_passes=False),
    )
    def hist(idx8_hbm, out_hbm, idx_v, hist_v):
        cc = lax.axis_index("c")
        ss = lax.axis_index("s")
        wid = cc * 16 + ss
        zero16 = jnp.zeros((16,), f32)
        ones16 = jnp.ones((16,), f32)
        for g in range(8):
            def zb(i, _):
                hist_v[pl.ds(i * 16, 16)] = zero16
                return 0
            lax.fori_loop(0, NB // 16, zb, 0)
            pltpu.sync_copy(idx8_hbm.at[g, wid], idx_v)

            def eb(j, _):
                for kk in range(C // 16):
                    idx = idx_v[j, pl.ds(kk * 16, 16)]
                    plsc.addupdate_scatter(hist_v, [idx], ones16)
                return 0
            lax.fori_loop(0, CT, eb, 0)
            pltpu.sync_copy(hist_v, out_hbm.at[wid, g])

    return hist


@functools.cache
def _apply_kernel(H, NV):
    @functools.partial(
        pl.kernel,
        out_type=jax.ShapeDtypeStruct((NV, 2, NP, H), f32),
        mesh=_MESH,
        scratch_types=[
            pltpu.VMEM((CT, C), i32),
            pltpu.VMEM((CT, C), i32),
            pltpu.VMEM((C, H), f32),
            pltpu.VMEM_SHARED((NP, H), f32),
            pltpu.SemaphoreType.DMA,
        ],
        compiler_params=pltpu.CompilerParams(use_tc_tiling_on_sc=False),
    )
    def apply(src_hbm, dst_hbm, hs_hbm, out_hbm, src_v, dst_v, rows_v,
              acc, sem):
        cc = lax.axis_index("c")
        ss = lax.axis_index("s")
        wid = cc * 16 + ss
        base = ss * RPT
        zero16 = jnp.zeros((16,), f32)

        for v in range(NV):
            def zrow(i, _):
                for kk in range(H // 16):
                    rows_v[i, pl.ds(kk * 16, 16)] = zero16
                return 0
            lax.fori_loop(0, C, zrow, 0)
            for b in range(RPT // C):
                pltpu.sync_copy(rows_v, acc.at[pl.ds(base + b * C, C)])

            pltpu.sync_copy(src_hbm.at[v, wid], src_v)
            pltpu.sync_copy(dst_hbm.at[v, wid], dst_v)
            plsc.subcore_barrier()

            def body(j, _):
                pltpu.async_copy(hs_hbm.at[v].at[src_v.at[j]],
                                 rows_v, sem).wait()
                pltpu.sync_copy(rows_v, acc.at[dst_v.at[j]], add=True)
                return 0
            lax.fori_loop(0, CT, body, 0)

            plsc.subcore_barrier()
            pltpu.sync_copy(acc.at[pl.ds(base, RPT)],
                            out_hbm.at[v, cc, pl.ds(base, RPT)])

    return apply


_BV = 1000


def _norms(hists):
    BB = 1280

    def body(h_ref, o_ref):
        v = h_ref[...]
        deg = v[0]
        for t in range(1, NT):
            deg = deg + v[t]
        row = lax.broadcasted_iota(i32, (8, BB), 0)
        deg = deg + jnp.where(row >= 6, 1.0, 0.0).astype(f32)
        o_ref[...] = jnp.where(deg > 0, lax.rsqrt(deg), 0.0)

    return pl.pallas_call(
        body,
        grid=(NB // BB,),
        in_specs=[pl.BlockSpec((NT, 8, BB), lambda i: (0, 0, i))],
        out_specs=pl.BlockSpec((8, BB), lambda i: (0, i)),
        out_shape=jax.ShapeDtypeStruct((8, NB), f32),
    )(hists)


def _mm_scale(x, W, ns):
    K, H2 = W.shape

    def body(x_ref, w_ref, ns_ref, o_ref):
        h = jnp.dot(x_ref[...], w_ref[...], preferred_element_type=f32,
                    precision=lax.Precision.HIGHEST)
        o_ref[...] = h * ns_ref[...]

    return pl.pallas_call(
        body,
        grid=(N // _BV,),
        in_specs=[pl.BlockSpec((_BV, K), lambda i: (i, 0)),
                  pl.BlockSpec((K, H2), lambda i: (0, 0)),
                  pl.BlockSpec((_BV, 1), lambda i: (i, 0))],
        out_specs=pl.BlockSpec((_BV, H2), lambda i: (i, 0)),
        out_shape=jax.ShapeDtypeStruct((N, H2), f32),
    )(x, W, ns)


def _combine(parts, extra, nd, b, act, W=None, ns=None):
    H = parts.shape[2]
    have_extra = extra is not None
    have_mm = W is not None

    def body(*refs):
        it = iter(refs)
        p_ref = next(it)
        e_ref = next(it) if have_extra else None
        nd_ref = next(it)
        b_ref = next(it)
        w_ref = next(it) if have_mm else None
        ns_ref = next(it) if have_mm else None
        o_ref = next(it)
        v = p_ref[...]
        agg = v[0] + v[1]
        if have_extra:
            agg = agg + e_ref[...]
        h = agg * nd_ref[...] + b_ref[...]
        if act:
            h = jnp.maximum(h, 0.0)
        if have_mm:
            h = jnp.dot(h, w_ref[...], preferred_element_type=f32,
                        precision=lax.Precision.HIGHEST) * ns_ref[...]
        o_ref[...] = h

    H2 = W.shape[1] if have_mm else H
    in_specs = [pl.BlockSpec((2, _BV, H), lambda i: (0, i, 0))]
    args = [parts]
    if have_extra:
        in_specs.append(pl.BlockSpec((_BV, H), lambda i: (i, 0)))
        args.append(extra)
    in_specs += [pl.BlockSpec((_BV, 1), lambda i: (i, 0)),
                 pl.BlockSpec((1, H), lambda i: (0, 0))]
    args += [nd, b.reshape(1, H)]
    if have_mm:
        in_specs += [pl.BlockSpec(W.shape, lambda i: (0, 0)),
                     pl.BlockSpec((_BV, 1), lambda i: (i, 0))]
        args += [W, ns]

    return pl.pallas_call(
        body,
        grid=(N // _BV,),
        in_specs=in_specs,
        out_specs=pl.BlockSpec((_BV, H2), lambda i: (i, 0)),
        out_shape=jax.ShapeDtypeStruct((N, H2), f32),
    )(*args)


def _fusion(h0, h1, h2, Wf1, Wf2, Wf3, WfcT, bfc, Wma, ns):
    H = h0.shape[1]

    def body(h0_ref, h1_ref, h2_ref, w1_ref, w2_ref, w3_ref, wc_ref, bc_ref,
             wm_ref, ns_ref, o_ref):
        kw = dict(preferred_element_type=f32, precision=lax.Precision.HIGHEST)
        y = (jnp.dot(h0_ref[...], w1_ref[...], **kw)
             + jnp.dot(h1_ref[...], w2_ref[...], **kw)
             + jnp.dot(h2_ref[...], w3_ref[...], **kw))
        xh0 = jnp.dot(y, wc_ref[...], **kw) + bc_ref[...]
        o_ref[...] = jnp.dot(xh0, wm_ref[...], **kw) * ns_ref[...]

    mat = pl.BlockSpec((H, H), lambda i: (0, 0))
    blk = pl.BlockSpec((_BV, H), lambda i: (i, 0))
    return pl.pallas_call(
        body,
        grid=(N // _BV,),
        in_specs=[blk, blk, blk, mat, mat, mat, mat,
                  pl.BlockSpec((1, H), lambda i: (0, 0)),
                  mat,
                  pl.BlockSpec((_BV, 1), lambda i: (i, 0))],
        out_specs=blk,
        out_shape=jax.ShapeDtypeStruct((N, H), f32),
    )(h0, h1, h2, Wf1, Wf2, Wf3, WfcT, bfc.reshape(1, H), Wma, ns)


def _decoder(xh, Wdec):
    BR = 200
    H = xh.shape[1]

    def body(xr_ref, xf_ref, w_ref, o_ref):
        kw = dict(preferred_element_type=f32, precision=lax.Precision.HIGHEST)
        t = jnp.dot(xr_ref[...], w_ref[...], **kw)
        logits = lax.dot_general(t, xf_ref[...], (((1,), (1,)), ((), ())), **kw)
        o_ref[...] = 1.0 / (1.0 + jnp.exp(-logits))

    return pl.pallas_call(
        body,
        grid=(N // BR,),
        in_specs=[pl.BlockSpec((BR, H), lambda i: (i, 0)),
                  pl.BlockSpec((N, H), lambda i: (0, 0)),
                  pl.BlockSpec((H, H), lambda i: (0, 0))],
        out_specs=pl.BlockSpec((BR, N), lambda i: (i, 0)),
        out_shape=jax.ShapeDtypeStruct((N, N), f32),
    )(xh, xh, Wdec)


def _prep_edges(g):
    src = jnp.concatenate([g[0].astype(i32), jnp.zeros((EPAD,), i32)])
    srch = jnp.concatenate([g[0].astype(i32), jnp.full((EPAD,), N, i32)])
    dst = jnp.concatenate([g[1].astype(i32), jnp.full((EPAD,), N, i32)])
    return (src.reshape(NT, CT, C), srch.reshape(NT, CT, C),
            dst.reshape(NT, CT, C))


def kernel(graph0, graph1, graph2, feature0, feature1, feature2, graph,
           W0a, b0a, W0b, b0b, W1a, b1a, W1b, b1b, W2a, b2a, W2b, b2b,
           Wma, bma, Wmb, bmb, Wf1, Wf2, Wf3, Wfc, bfc, Wdec):
    s0, sh0, d0 = _prep_edges(graph0)
    s1, sh1, d1 = _prep_edges(graph1)
    s2, sh2, d2 = _prep_edges(graph2)
    sm, shm, dm = _prep_edges(graph)

    idx8 = jnp.stack([sh0, d0, sh1, d1, sh2, d2, shm, dm])
    norms = _norms(_hist_kernel()(idx8))

    def col(g):
        return norms[g, :N].reshape(N, 1)

    ns0, nd0, ns1, nd1, ns2, nd2, nsm, ndm = (col(g) for g in range(8))

    ap128 = _apply_kernel(128, 3)
    ap64v = _apply_kernel(64, 3)
    ap64m = _apply_kernel(64, 1)
    ap32m = _apply_kernel(32, 1)

    src3 = jnp.stack([s0, s1, s2])
    dst3 = jnp.stack([d0, d1, d2])

    hs_a = jnp.stack([_mm_scale(feature0, W0a, ns0),
                      _mm_scale(feature1, W1a, ns1),
                      _mm_scale(feature2, W2a, ns2)])
    parts_a = ap128(src3, dst3, hs_a)

    hs_b = jnp.stack([
        _combine(parts_a[0], None, nd0, b0a, True, W=W0b, ns=ns0),
        _combine(parts_a[1], None, nd1, b1a, True, W=W1b, ns=ns1),
        _combine(parts_a[2], None, nd2, b2a, True, W=W2b, ns=ns2),
    ])
    parts_b = ap64v(src3, dst3, hs_b)

    h0 = _combine(parts_b[0], None, nd0, b0b, False)
    h1 = _combine(parts_b[1], None, nd1, b1b, False)
    h2 = _combine(parts_b[2], None, nd2, b2b, False)

    hs_m = _fusion(h0, h1, h2, Wf1, Wf2, Wf3, Wfc.T, bfc, Wma, nsm)
    parts_m = ap64m(sm[None], dm[None], hs_m[None])
    hs2_m = _combine(parts_m[0], hs_m, ndm, bma, True, W=Wmb, ns=nsm)
    parts2_m = ap32m(sm[None], dm[None], hs2_m[None])
    xh = _combine(parts2_m[0], hs2_m, ndm, bmb, False)

    adj0 = _decoder(xh, Wdec)
    return adj0, xh


# speedup vs baseline: 1.5061x; 1.1711x over previous
import functools

import jax
import jax.numpy as jnp
from jax import lax
from jax.experimental import pallas as pl
from jax.experimental.pallas import tpu as pltpu
from jax.experimental.pallas import tpu_sc as plsc

N = 10000
E = 320000
NP = 10240
C = 128
NT = 32
CT = (E + NT * C - 1) // (NT * C)
EPAD = NT * CT * C - E
EPT = CT * C
RPT = NP // 16
NB = NP

f32 = jnp.float32
i32 = jnp.int32

_MESH = plsc.VectorSubcoreMesh(core_axis_name="c", subcore_axis_name="s")


@functools.cache
def _hist_kernel():
    @functools.partial(
        pl.kernel,
        out_type=jax.ShapeDtypeStruct((NT, 8, NB), f32),
        mesh=_MESH,
        scratch_types=[
            pltpu.VMEM((CT, C), i32),
            pltpu.VMEM((NB,), f32),
        ],
        compiler_params=pltpu.CompilerParams(needs_layout_passes=False),
    )
    def hist(idx8_hbm, out_hbm, idx_v, hist_v):
        cc = lax.axis_index("c")
        ss = lax.axis_index("s")
        wid = cc * 16 + ss
        zero16 = jnp.zeros((16,), f32)
        ones16 = jnp.ones((16,), f32)
        for g in range(8):
            def zb(i, _):
                hist_v[pl.ds(i * 16, 16)] = zero16
                return 0
            lax.fori_loop(0, NB // 16, zb, 0)
            pltpu.sync_copy(idx8_hbm.at[g, wid], idx_v)

            def eb(j, _):
                for kk in range(C // 16):
                    idx = idx_v[j, pl.ds(kk * 16, 16)]
                    plsc.addupdate_scatter(hist_v, [idx], ones16)
                return 0
            lax.fori_loop(0, CT, eb, 0)
            pltpu.sync_copy(hist_v, out_hbm.at[wid, g])

    return hist


@functools.cache
def _apply_kernel(H, NV):
    CC = 64
    CN = EPT // CC

    @functools.partial(
        pl.kernel,
        out_type=jax.ShapeDtypeStruct((NV, 2, NP, H), f32),
        mesh=_MESH,
        scratch_types=[
            pltpu.VMEM((CN, CC), i32),
            pltpu.VMEM((CN, CC), i32),
            pltpu.VMEM((CC, H), f32),
            pltpu.VMEM((CC, H), f32),
            pltpu.VMEM_SHARED((NP, H), f32),
            pltpu.SemaphoreType.DMA,
            pltpu.SemaphoreType.DMA,
        ],
        compiler_params=pltpu.CompilerParams(use_tc_tiling_on_sc=False),
    )
    def apply(src_hbm, dst_hbm, hs_hbm, out_hbm, src_v, dst_v, r0, r1,
              acc, s0, s1):
        cc = lax.axis_index("c")
        ss = lax.axis_index("s")
        wid = cc * 16 + ss
        base = ss * RPT
        zero16 = jnp.zeros((16,), f32)

        def gather(j, rbuf, sem):
            pltpu.async_copy(hs_hbm.at[v].at[src_v.at[j]], rbuf, sem)

        def scatter(j, rbuf, sem):
            pltpu.make_async_copy(hs_hbm.at[v].at[pl.ds(0, CC)],
                                  rbuf, sem).wait()
            pltpu.sync_copy(rbuf, acc.at[dst_v.at[j]], add=True)

        for v in range(NV):
            def zrow(i, _):
                for kk in range(H // 16):
                    r0[i, pl.ds(kk * 16, 16)] = zero16
                return 0
            lax.fori_loop(0, CC, zrow, 0)
            for b in range(RPT // CC):
                pltpu.sync_copy(r0, acc.at[pl.ds(base + b * CC, CC)])

            pltpu.sync_copy(src_hbm.at[v, wid], src_v)
            pltpu.sync_copy(dst_hbm.at[v, wid], dst_v)
            gather(0, r0, s0)
            plsc.subcore_barrier()

            def body(k, _):
                j = k * 2
                gather(j + 1, r1, s1)
                scatter(j, r0, s0)
                gather(j + 2, r0, s0)
                scatter(j + 1, r1, s1)
                return 0
            lax.fori_loop(0, CN // 2 - 1, body, 0)

            gather(CN - 1, r1, s1)
            scatter(CN - 2, r0, s0)
            scatter(CN - 1, r1, s1)

            plsc.subcore_barrier()
            pltpu.sync_copy(acc.at[pl.ds(base, RPT)],
                            out_hbm.at[v, cc, pl.ds(base, RPT)])

    return apply


_BV = 1000


def _norms(hists):
    BB = 1280

    def body(h_ref, o_ref):
        v = h_ref[...]
        deg = v[0]
        for t in range(1, NT):
            deg = deg + v[t]
        row = lax.broadcasted_iota(i32, (8, BB), 0)
        deg = deg + jnp.where(row >= 6, 1.0, 0.0).astype(f32)
        o_ref[...] = jnp.where(deg > 0, lax.rsqrt(deg), 0.0)

    return pl.pallas_call(
        body,
        grid=(NB // BB,),
        in_specs=[pl.BlockSpec((NT, 8, BB), lambda i: (0, 0, i))],
        out_specs=pl.BlockSpec((8, BB), lambda i: (0, i)),
        out_shape=jax.ShapeDtypeStruct((8, NB), f32),
    )(hists)


def _mm_scale(x, W, ns):
    K, H2 = W.shape

    def body(x_ref, w_ref, ns_ref, o_ref):
        h = jnp.dot(x_ref[...], w_ref[...], preferred_element_type=f32,
                    precision=lax.Precision.HIGHEST)
        o_ref[...] = h * ns_ref[...]

    return pl.pallas_call(
        body,
        grid=(N // _BV,),
        in_specs=[pl.BlockSpec((_BV, K), lambda i: (i, 0)),
                  pl.BlockSpec((K, H2), lambda i: (0, 0)),
                  pl.BlockSpec((_BV, 1), lambda i: (i, 0))],
        out_specs=pl.BlockSpec((_BV, H2), lambda i: (i, 0)),
        out_shape=jax.ShapeDtypeStruct((N, H2), f32),
    )(x, W, ns)


def _combine(parts, extra, nd, b, act, W=None, ns=None):
    H = parts.shape[2]
    have_extra = extra is not None
    have_mm = W is not None

    def body(*refs):
        it = iter(refs)
        p_ref = next(it)
        e_ref = next(it) if have_extra else None
        nd_ref = next(it)
        b_ref = next(it)
        w_ref = next(it) if have_mm else None
        ns_ref = next(it) if have_mm else None
        o_ref = next(it)
        v = p_ref[...]
        agg = v[0] + v[1]
        if have_extra:
            agg = agg + e_ref[...]
        h = agg * nd_ref[...] + b_ref[...]
        if act:
            h = jnp.maximum(h, 0.0)
        if have_mm:
            h = jnp.dot(h, w_ref[...], preferred_element_type=f32,
                        precision=lax.Precision.HIGHEST) * ns_ref[...]
        o_ref[...] = h

    H2 = W.shape[1] if have_mm else H
    in_specs = [pl.BlockSpec((2, _BV, H), lambda i: (0, i, 0))]
    args = [parts]
    if have_extra:
        in_specs.append(pl.BlockSpec((_BV, H), lambda i: (i, 0)))
        args.append(extra)
    in_specs += [pl.BlockSpec((_BV, 1), lambda i: (i, 0)),
                 pl.BlockSpec((1, H), lambda i: (0, 0))]
    args += [nd, b.reshape(1, H)]
    if have_mm:
        in_specs += [pl.BlockSpec(W.shape, lambda i: (0, 0)),
                     pl.BlockSpec((_BV, 1), lambda i: (i, 0))]
        args += [W, ns]

    return pl.pallas_call(
        body,
        grid=(N // _BV,),
        in_specs=in_specs,
        out_specs=pl.BlockSpec((_BV, H2), lambda i: (i, 0)),
        out_shape=jax.ShapeDtypeStruct((N, H2), f32),
    )(*args)


def _fusion(h0, h1, h2, Wf1, Wf2, Wf3, WfcT, bfc, Wma, ns):
    H = h0.shape[1]

    def body(h0_ref, h1_ref, h2_ref, w1_ref, w2_ref, w3_ref, wc_ref, bc_ref,
             wm_ref, ns_ref, o_ref):
        kw = dict(preferred_element_type=f32, precision=lax.Precision.HIGHEST)
        y = (jnp.dot(h0_ref[...], w1_ref[...], **kw)
             + jnp.dot(h1_ref[...], w2_ref[...], **kw)
             + jnp.dot(h2_ref[...], w3_ref[...], **kw))
        xh0 = jnp.dot(y, wc_ref[...], **kw) + bc_ref[...]
        o_ref[...] = jnp.dot(xh0, wm_ref[...], **kw) * ns_ref[...]

    mat = pl.BlockSpec((H, H), lambda i: (0, 0))
    blk = pl.BlockSpec((_BV, H), lambda i: (i, 0))
    return pl.pallas_call(
        body,
        grid=(N // _BV,),
        in_specs=[blk, blk, blk, mat, mat, mat, mat,
                  pl.BlockSpec((1, H), lambda i: (0, 0)),
                  mat,
                  pl.BlockSpec((_BV, 1), lambda i: (i, 0))],
        out_specs=blk,
        out_shape=jax.ShapeDtypeStruct((N, H), f32),
    )(h0, h1, h2, Wf1, Wf2, Wf3, WfcT, bfc.reshape(1, H), Wma, ns)


def _decoder(xh, Wdec):
    BR = 200
    H = xh.shape[1]

    def body(xr_ref, xf_ref, w_ref, o_ref):
        t = jnp.dot(xr_ref[...], w_ref[...], preferred_element_type=f32,
                    precision=lax.Precision.HIGHEST)
        bf16 = jnp.bfloat16
        xf = xf_ref[...]
        t_hi = t.astype(bf16)
        xf_hi = xf.astype(bf16)
        t_lo = (t - t_hi.astype(f32)).astype(bf16)
        xf_lo = (xf - xf_hi.astype(f32)).astype(bf16)
        dims = (((1,), (1,)), ((), ()))
        kw = dict(preferred_element_type=f32)
        logits = (lax.dot_general(t_hi, xf_hi, dims, **kw)
                  + lax.dot_general(t_hi, xf_lo, dims, **kw)
                  + lax.dot_general(t_lo, xf_hi, dims, **kw))
        o_ref[...] = 0.5 * (1.0 + jnp.tanh(0.5 * logits))

    return pl.pallas_call(
        body,
        grid=(N // BR,),
        in_specs=[pl.BlockSpec((BR, H), lambda i: (i, 0)),
                  pl.BlockSpec((N, H), lambda i: (0, 0)),
                  pl.BlockSpec((H, H), lambda i: (0, 0))],
        out_specs=pl.BlockSpec((BR, N), lambda i: (i, 0)),
        out_shape=jax.ShapeDtypeStruct((N, N), f32),
    )(xh, xh, Wdec)


def _prep_edges(g):
    src = jnp.concatenate([g[0].astype(i32), jnp.zeros((EPAD,), i32)])
    srch = jnp.concatenate([g[0].astype(i32), jnp.full((EPAD,), N, i32)])
    dst = jnp.concatenate([g[1].astype(i32), jnp.full((EPAD,), N, i32)])
    return (src.reshape(NT, CT, C), srch.reshape(NT, CT, C),
            dst.reshape(NT, CT, C))


def kernel(graph0, graph1, graph2, feature0, feature1, feature2, graph,
           W0a, b0a, W0b, b0b, W1a, b1a, W1b, b1b, W2a, b2a, W2b, b2b,
           Wma, bma, Wmb, bmb, Wf1, Wf2, Wf3, Wfc, bfc, Wdec):
    s0, sh0, d0 = _prep_edges(graph0)
    s1, sh1, d1 = _prep_edges(graph1)
    s2, sh2, d2 = _prep_edges(graph2)
    sm, shm, dm = _prep_edges(graph)

    idx8 = jnp.stack([sh0, d0, sh1, d1, sh2, d2, shm, dm])
    norms = _norms(_hist_kernel()(idx8))

    def col(g):
        return norms[g, :N].reshape(N, 1)

    ns0, nd0, ns1, nd1, ns2, nd2, nsm, ndm = (col(g) for g in range(8))

    ap128 = _apply_kernel(128, 3)
    ap64v = _apply_kernel(64, 3)
    ap64m = _apply_kernel(64, 1)
    ap32m = _apply_kernel(32, 1)

    def shp(e, H):
        CC = 64
        return e.reshape(e.shape[0], NT, EPT // CC, CC)

    src3 = jnp.stack([s0, s1, s2])
    dst3 = jnp.stack([d0, d1, d2])
    smc = sm[None]
    dmc = dm[None]

    hs_a = jnp.stack([_mm_scale(feature0, W0a, ns0),
                      _mm_scale(feature1, W1a, ns1),
                      _mm_scale(feature2, W2a, ns2)])
    parts_a = ap128(shp(src3, 128), shp(dst3, 128), hs_a)

    hs_b = jnp.stack([
        _combine(parts_a[0], None, nd0, b0a, True, W=W0b, ns=ns0),
        _combine(parts_a[1], None, nd1, b1a, True, W=W1b, ns=ns1),
        _combine(parts_a[2], None, nd2, b2a, True, W=W2b, ns=ns2),
    ])
    parts_b = ap64v(shp(src3, 64), shp(dst3, 64), hs_b)

    h0 = _combine(parts_b[0], None, nd0, b0b, False)
    h1 = _combine(parts_b[1], None, nd1, b1b, False)
    h2 = _combine(parts_b[2], None, nd2, b2b, False)

    hs_m = _fusion(h0, h1, h2, Wf1, Wf2, Wf3, Wfc.T, bfc, Wma, nsm)
    parts_m = ap64m(shp(smc, 64), shp(dmc, 64), hs_m[None])
    hs2_m = _combine(parts_m[0], hs_m, ndm, bma, True, W=Wmb, ns=nsm)
    parts2_m = ap32m(shp(smc, 32), shp(dmc, 32), hs2_m[None])
    xh = _combine(parts2_m[0], hs2_m, ndm, bmb, False)

    adj0 = _decoder(xh, Wdec)
    return adj0, xh
